# baseline (speedup 1.0000x reference)
import functools

import jax
import jax.numpy as jnp
from jax.experimental import pallas as pl
from jax.experimental.pallas import tpu as pltpu

RADIUS = 2
K_STEPS = 3
ETA = 0.1
MU_MAX = 10.0
HALO = 16
BLK_T = 1024


def _gelu_exact(x):
    return 0.5 * x * (1.0 + jax.lax.erf(x * 0.7071067811865476))


def _softplus(x):
    return jnp.log1p(jnp.exp(-jnp.abs(x))) + jnp.maximum(x, 0.0)


def _shift_up(a, n):
    z = jnp.zeros((n,) + a.shape[1:], a.dtype)
    return jnp.concatenate([a[n:], z], axis=0)


def _roll_up(a, n):
    return jnp.roll(a, -n, axis=0)


def _roll_down(a, n):
    return jnp.roll(a, n, axis=0)


def _shift_down(a, n):
    z = jnp.zeros((n,) + a.shape[1:], a.dtype)
    return jnp.concatenate([z, a[:-n]], axis=0)


def _friction_kernel(hid_any, w1_ref, b1_ref, w2row_ref, b2_ref, wq_ref,
                     bq_ref, gamma_ref, beta_ref, out_ref, eng_ref,
                     h_scr, copy_sems, *, L, T, W, NBLK):
    b = pl.program_id(0)
    blk = pl.program_id(1)
    slot = jax.lax.rem(blk, 2)

    def start_copy(kk, sl):
        wsn = pl.multiple_of(jnp.clip(kk * T - HALO, 0, L - W), 8)
        pltpu.make_async_copy(hid_any.at[b, pl.ds(wsn, W), :],
                              h_scr.at[sl], copy_sems.at[sl]).start()

    @pl.when(blk == 0)
    def _():
        start_copy(0, 0)

    @pl.when(blk + 1 < NBLK)
    def _():
        start_copy(blk + 1, 1 - slot)

    ws = pl.multiple_of(jnp.clip(blk * T - HALO, 0, L - W), 8)
    loc0 = blk * T - ws
    pltpu.make_async_copy(hid_any.at[b, pl.ds(ws, W), :],
                          h_scr.at[slot], copy_sems.at[slot]).wait()
    h = h_scr[slot]

    q = jnp.dot(h, wq_ref[...], preferred_element_type=jnp.float32)
    Q = ETA * (q + bq_ref[...])

    gidx = ws + jax.lax.broadcasted_iota(jnp.int32, (W, 1), 0)
    n2 = jnp.sum(h * h, axis=1, keepdims=True)
    ni = jnp.maximum(jnp.sqrt(n2), 1e-6)

    def edge_mu(r):
        dot = jnp.sum(h * _shift_up(h, r), axis=1, keepdims=True)
        d2 = jnp.maximum(n2 + _shift_up(n2, r) - 2.0 * dot, 0.0)
        dist = jnp.sqrt(d2)
        cos = dot / (ni * _shift_up(ni, r))
        hm = _gelu_exact(dist * w1_ref[0:1, :] + cos * w1_ref[1:2, :]
                         + b1_ref[...])
        mu = jnp.sum(hm * w2row_ref[...], axis=1, keepdims=True) + b2_ref[0, 0]
        mu = jnp.minimum(_softplus(mu) + 1e-5, MU_MAX)
        return jnp.where(gidx <= L - 1 - r, mu, 0.0)

    w1e = edge_mu(1)
    w2e = edge_mu(2)

    deg = w1e + _shift_down(w1e, 1) + w2e + _shift_down(w2e, 2)
    inv = jax.lax.rsqrt(jnp.maximum(deg, 1e-6))
    ec1 = (ETA * w1e) * inv * _shift_up(inv, 1)
    ec2 = (ETA * w2e) * inv * _shift_up(inv, 2)
    ec1d = _shift_down(ec1, 1)
    ec2d = _shift_down(ec2, 2)
    A = 1.0 - (ec1 + ec1d + ec2 + ec2d)

    s = h
    for _ in range(K_STEPS):
        acc = Q + A * s
        acc = acc + ec1 * _shift_up(s, 1)
        acc = acc + ec1d * _shift_down(s, 1)
        acc = acc + ec2 * _shift_up(s, 2)
        s = acc + ec2d * _shift_down(s, 2)

    d1 = s - _shift_up(s, 1)
    d2 = s - _shift_up(s, 2)
    en1 = jnp.sum(d1 * d1, axis=1, keepdims=True)
    en2 = jnp.sum(d2 * d2, axis=1, keepdims=True)
    own = (gidx >= blk * T) & (gidx < blk * T + T)
    contrib = 0.5 * jnp.sum(jnp.where(own, w1e * en1 + w2e * en2, 0.0))
    prev = jnp.where(blk == 0, 0.0, eng_ref[0, 0, 0])
    eng_ref[...] = jnp.full_like(eng_ref, prev + contrib)

    res = s + h
    mean = jnp.mean(res, axis=1, keepdims=True)
    cen = res - mean
    var = jnp.mean(cen * cen, axis=1, keepdims=True)
    norm = cen * jax.lax.rsqrt(var + 1e-5) * gamma_ref[...] + beta_ref[...]
    if W == T:
        out_ref[0, :, :] = norm
    else:
        for off in (0, HALO, 2 * HALO):
            @pl.when(loc0 == off)
            def _(off=off):
                out_ref[0, :, :] = norm[off:off + T]


def kernel(hidden, attention_mask, W1, b1, W2, b2, Wq, bq, gamma, beta):
    del attention_mask
    B, L, H = hidden.shape
    T = BLK_T if L % BLK_T == 0 and L > BLK_T else L
    W = T + 2 * HALO if L > T else T
    nblk = L // T
    inner = W1.shape[1]

    f32 = jnp.float32
    grid = (B, nblk)
    out_shape = (
        jax.ShapeDtypeStruct((B, L, H), f32),
        jax.ShapeDtypeStruct((B, 8, 128), f32),
    )
    kern = functools.partial(_friction_kernel, L=L, T=T, W=W, NBLK=nblk)
    out, eng = pl.pallas_call(
        kern,
        grid=grid,
        in_specs=[
            pl.BlockSpec(memory_space=pl.ANY),
            pl.BlockSpec((2, inner), lambda b, i: (0, 0)),
            pl.BlockSpec((1, inner), lambda b, i: (0, 0)),
            pl.BlockSpec((1, inner), lambda b, i: (0, 0)),
            pl.BlockSpec((1, 1), lambda b, i: (0, 0)),
            pl.BlockSpec((H, H), lambda b, i: (0, 0)),
            pl.BlockSpec((1, H), lambda b, i: (0, 0)),
            pl.BlockSpec((1, H), lambda b, i: (0, 0)),
            pl.BlockSpec((1, H), lambda b, i: (0, 0)),
        ],
        out_specs=(
            pl.BlockSpec((1, T, H), lambda b, i: (b, i, 0)),
            pl.BlockSpec((1, 8, 128), lambda b, i: (b, 0, 0)),
        ),
        out_shape=out_shape,
        scratch_shapes=[
            pltpu.VMEM((2, W, H), f32),
            pltpu.SemaphoreType.DMA((2,)),
        ],
        compiler_params=pltpu.CompilerParams(
            dimension_semantics=("parallel", "arbitrary"),
        ),
    )(
        hidden.astype(f32),
        W1.astype(f32),
        b1.reshape(1, inner).astype(f32),
        W2.reshape(1, inner).astype(f32),
        b2.reshape(1, 1).astype(f32),
        Wq.astype(f32),
        bq.reshape(1, H).astype(f32),
        gamma.reshape(1, H).astype(f32),
        beta.reshape(1, H).astype(f32),
    )
    return out, eng[:, 0, 0]


# speedup vs baseline: 1.0130x; 1.0130x over previous
import functools

import jax
import jax.numpy as jnp
from jax.experimental import pallas as pl
from jax.experimental.pallas import tpu as pltpu

RADIUS = 2
K_STEPS = 3
ETA = 0.1
MU_MAX = 10.0
HALO = 16
BLK_T = 1024


def _gelu_exact(x):
    return 0.5 * x * (1.0 + jax.lax.erf(x * 0.7071067811865476))


def _softplus(x):
    return jnp.log1p(jnp.exp(-jnp.abs(x))) + jnp.maximum(x, 0.0)


def _shift_up(a, n):
    z = jnp.zeros((n,) + a.shape[1:], a.dtype)
    return jnp.concatenate([a[n:], z], axis=0)


def _roll_up(a, n):
    return jnp.roll(a, -n, axis=0)


def _roll_down(a, n):
    return jnp.roll(a, n, axis=0)


def _shift_down(a, n):
    z = jnp.zeros((n,) + a.shape[1:], a.dtype)
    return jnp.concatenate([z, a[:-n]], axis=0)


def _friction_kernel(hid_any, w1_ref, b1_ref, w2row_ref, b2_ref, wq_ref,
                     bq_ref, gamma_ref, beta_ref, out_ref, eng_ref,
                     h_scr, copy_sems, *, L, T, W, NBLK):
    b = pl.program_id(0)
    blk = pl.program_id(1)
    g = b * NBLK + blk
    G = pl.num_programs(0) * NBLK
    slot = jax.lax.rem(g, 2)

    def start_copy(gg, sl):
        bb = gg // NBLK
        kk = gg - bb * NBLK
        wsn = pl.multiple_of(jnp.clip(kk * T - HALO, 0, L - W), 8)
        pltpu.make_async_copy(hid_any.at[bb, pl.ds(wsn, W), :],
                              h_scr.at[sl], copy_sems.at[sl]).start()

    @pl.when(g == 0)
    def _():
        start_copy(0, 0)

    @pl.when(g + 1 < G)
    def _():
        start_copy(g + 1, 1 - slot)

    ws = pl.multiple_of(jnp.clip(blk * T - HALO, 0, L - W), 8)
    loc0 = blk * T - ws
    pltpu.make_async_copy(hid_any.at[b, pl.ds(ws, W), :],
                          h_scr.at[slot], copy_sems.at[slot]).wait()
    h = h_scr[slot]

    q = jnp.dot(h, wq_ref[...], preferred_element_type=jnp.float32)
    Q = ETA * (q + bq_ref[...])

    gidx = ws + jax.lax.broadcasted_iota(jnp.int32, (W, 1), 0)
    n2 = jnp.sum(h * h, axis=1, keepdims=True)
    ni = jnp.maximum(jnp.sqrt(n2), 1e-6)

    def edge_mu(r):
        dot = jnp.sum(h * _shift_up(h, r), axis=1, keepdims=True)
        d2 = jnp.maximum(n2 + _shift_up(n2, r) - 2.0 * dot, 0.0)
        dist = jnp.sqrt(d2)
        cos = dot / (ni * _shift_up(ni, r))
        hm = _gelu_exact(dist * w1_ref[0:1, :] + cos * w1_ref[1:2, :]
                         + b1_ref[...])
        mu = jnp.sum(hm * w2row_ref[...], axis=1, keepdims=True) + b2_ref[0, 0]
        mu = jnp.minimum(_softplus(mu) + 1e-5, MU_MAX)
        return jnp.where(gidx <= L - 1 - r, mu, 0.0)

    w1e = edge_mu(1)
    w2e = edge_mu(2)

    deg = w1e + _shift_down(w1e, 1) + w2e + _shift_down(w2e, 2)
    inv = jax.lax.rsqrt(jnp.maximum(deg, 1e-6))
    ec1 = (ETA * w1e) * inv * _shift_up(inv, 1)
    ec2 = (ETA * w2e) * inv * _shift_up(inv, 2)
    ec1d = _shift_down(ec1, 1)
    ec2d = _shift_down(ec2, 2)
    A = 1.0 - (ec1 + ec1d + ec2 + ec2d)

    s = h
    for _ in range(K_STEPS):
        acc = Q + A * s
        acc = acc + ec1 * _shift_up(s, 1)
        acc = acc + ec1d * _shift_down(s, 1)
        acc = acc + ec2 * _shift_up(s, 2)
        s = acc + ec2d * _shift_down(s, 2)

    d1 = s - _shift_up(s, 1)
    d2 = s - _shift_up(s, 2)
    en1 = jnp.sum(d1 * d1, axis=1, keepdims=True)
    en2 = jnp.sum(d2 * d2, axis=1, keepdims=True)
    own = (gidx >= blk * T) & (gidx < blk * T + T)
    contrib = 0.5 * jnp.sum(jnp.where(own, w1e * en1 + w2e * en2, 0.0))
    prev = jnp.where(blk == 0, 0.0, eng_ref[0, 0, 0])
    eng_ref[...] = jnp.full_like(eng_ref, prev + contrib)

    res = s + h
    mean = jnp.mean(res, axis=1, keepdims=True)
    cen = res - mean
    var = jnp.mean(cen * cen, axis=1, keepdims=True)
    norm = cen * jax.lax.rsqrt(var + 1e-5) * gamma_ref[...] + beta_ref[...]
    if W == T:
        out_ref[0, :, :] = norm
    else:
        for off in (0, HALO, 2 * HALO):
            @pl.when(loc0 == off)
            def _(off=off):
                out_ref[0, :, :] = norm[off:off + T]


def kernel(hidden, attention_mask, W1, b1, W2, b2, Wq, bq, gamma, beta):
    del attention_mask
    B, L, H = hidden.shape
    T = BLK_T if L % BLK_T == 0 and L > BLK_T else L
    W = T + 2 * HALO if L > T else T
    nblk = L // T
    inner = W1.shape[1]

    f32 = jnp.float32
    grid = (B, nblk)
    out_shape = (
        jax.ShapeDtypeStruct((B, L, H), f32),
        jax.ShapeDtypeStruct((B, 8, 128), f32),
    )
    kern = functools.partial(_friction_kernel, L=L, T=T, W=W, NBLK=nblk)
    out, eng = pl.pallas_call(
        kern,
        grid=grid,
        in_specs=[
            pl.BlockSpec(memory_space=pl.ANY),
            pl.BlockSpec((2, inner), lambda b, i: (0, 0)),
            pl.BlockSpec((1, inner), lambda b, i: (0, 0)),
            pl.BlockSpec((1, inner), lambda b, i: (0, 0)),
            pl.BlockSpec((1, 1), lambda b, i: (0, 0)),
            pl.BlockSpec((H, H), lambda b, i: (0, 0)),
            pl.BlockSpec((1, H), lambda b, i: (0, 0)),
            pl.BlockSpec((1, H), lambda b, i: (0, 0)),
            pl.BlockSpec((1, H), lambda b, i: (0, 0)),
        ],
        out_specs=(
            pl.BlockSpec((1, T, H), lambda b, i: (b, i, 0)),
            pl.BlockSpec((1, 8, 128), lambda b, i: (b, 0, 0)),
        ),
        out_shape=out_shape,
        scratch_shapes=[
            pltpu.VMEM((2, W, H), f32),
            pltpu.SemaphoreType.DMA((2,)),
        ],
        compiler_params=pltpu.CompilerParams(
            dimension_semantics=("arbitrary", "arbitrary"),
        ),
    )(
        hidden.astype(f32),
        W1.astype(f32),
        b1.reshape(1, inner).astype(f32),
        W2.reshape(1, inner).astype(f32),
        b2.reshape(1, 1).astype(f32),
        Wq.astype(f32),
        bq.reshape(1, H).astype(f32),
        gamma.reshape(1, H).astype(f32),
        beta.reshape(1, H).astype(f32),
    )
    return out, eng[:, 0, 0]


# speedup vs baseline: 1.3192x; 1.3022x over previous
import functools

import jax
import jax.numpy as jnp
from jax.experimental import pallas as pl
from jax.experimental.pallas import tpu as pltpu

RADIUS = 2
K_STEPS = 3
ETA = 0.1
MU_MAX = 10.0
HALO = 32
BLK_T = 2048


def _gelu_exact(x):
    return 0.5 * x * (1.0 + jax.lax.erf(x * 0.7071067811865476))


def _softplus(x):
    return jnp.log1p(jnp.exp(-jnp.abs(x))) + jnp.maximum(x, 0.0)


def _shup(a):
    z = jnp.zeros((1,) + a.shape[1:], a.dtype)
    return jnp.concatenate([a[1:], z], axis=0)


def _shdn(a):
    z = jnp.zeros((1,) + a.shape[1:], a.dtype)
    return jnp.concatenate([z, a[:-1]], axis=0)


def _su1(xs):
    return xs[1:] + [_shup(xs[0])]


def _su2(xs):
    return xs[2:] + [_shup(xs[0]), _shup(xs[1])]


def _sd1(xs):
    return [_shdn(xs[7])] + xs[:7]


def _sd2(xs):
    return [_shdn(xs[6]), _shdn(xs[7])] + xs[:6]


def _friction_kernel(hid4_any, w1_ref, b1_ref, w2row_ref, b2_ref, wq_ref,
                     bq_ref, gamma_ref, beta_ref, out4_any, eng_ref,
                     h_scr, o_scr, in_sems, out_sems, *, L, T, W, NBLK):
    WP, TP = W // 8, T // 8
    b = pl.program_id(0)
    blk = pl.program_id(1)
    g = b * NBLK + blk
    G = pl.num_programs(0) * NBLK
    slot = jax.lax.rem(g, 2)
    H = wq_ref.shape[0]

    def in_copies(gg, sl):
        bb = gg // NBLK
        kk = gg - bb * NBLK
        wsp = jnp.clip(kk * T - HALO, 0, L - W) // 8
        return [pltpu.make_async_copy(
            hid4_any.at[bb, pl.ds(wsp, WP), r, :],
            h_scr.at[sl, r], in_sems.at[sl]) for r in range(8)]

    def out_copies(gg, sl):
        bb = gg // NBLK
        kk = gg - bb * NBLK
        return [pltpu.make_async_copy(
            o_scr.at[sl, r],
            out4_any.at[bb, pl.ds(kk * TP, TP), r, :],
            out_sems.at[sl]) for r in range(8)]

    @pl.when(g == 0)
    def _():
        for c in in_copies(0, 0):
            c.start()

    @pl.when(g + 1 < G)
    def _():
        for c in in_copies(g + 1, 1 - slot):
            c.start()

    @pl.when(g >= 2)
    def _():
        for c in out_copies(g - 2, slot):
            c.wait()

    for c in in_copies(g, slot):
        c.wait()

    ws = pl.multiple_of(jnp.clip(blk * T - HALO, 0, L - W), 8)
    loc0p = (blk * T - ws) // 8
    h = [h_scr[slot, r] for r in range(8)]

    Q = [ETA * (jnp.dot(h[r], wq_ref[...], preferred_element_type=jnp.float32)
                + bq_ref[...]) for r in range(8)]

    gidx = [ws + r + 8 * jax.lax.broadcasted_iota(jnp.int32, (WP, 1), 0)
            for r in range(8)]
    n2 = [jnp.sum(h[r] * h[r], axis=1, keepdims=True) for r in range(8)]
    ni = [jnp.maximum(jnp.sqrt(x), 1e-6) for x in n2]
    hs1, hs2 = _su1(h), _su2(h)
    n2s1, n2s2 = _su1(n2), _su2(n2)
    nis1, nis2 = _su1(ni), _su2(ni)

    def edge_mu(r, h_j, n2_j, ni_j, rr):
        dot = jnp.sum(h[r] * h_j, axis=1, keepdims=True)
        d2 = jnp.maximum(n2[r] + n2_j - 2.0 * dot, 0.0)
        dist = jnp.sqrt(d2)
        cos = dot / (ni[r] * ni_j)
        hm = _gelu_exact(dist * w1_ref[0:1, :] + cos * w1_ref[1:2, :]
                         + b1_ref[...])
        mu = jnp.sum(hm * w2row_ref[...], axis=1, keepdims=True) + b2_ref[0, 0]
        mu = jnp.minimum(_softplus(mu) + 1e-5, MU_MAX)
        return jnp.where(gidx[r] <= L - 1 - rr, mu, 0.0)

    w1e = [edge_mu(r, hs1[r], n2s1[r], nis1[r], 1) for r in range(8)]
    w2e = [edge_mu(r, hs2[r], n2s2[r], nis2[r], 2) for r in range(8)]

    w1d, w2d = _sd1(w1e), _sd2(w2e)
    deg = [w1e[r] + w1d[r] + w2e[r] + w2d[r] for r in range(8)]
    inv = [jax.lax.rsqrt(jnp.maximum(d, 1e-6)) for d in deg]
    invs1, invs2 = _su1(inv), _su2(inv)
    ec1 = [(ETA * w1e[r]) * inv[r] * invs1[r] for r in range(8)]
    ec2 = [(ETA * w2e[r]) * inv[r] * invs2[r] for r in range(8)]
    ec1d, ec2d = _sd1(ec1), _sd2(ec2)
    A = [1.0 - (ec1[r] + ec1d[r] + ec2[r] + ec2d[r]) for r in range(8)]

    s = h
    for _ in range(K_STEPS):
        ss1, ss2 = _su1(s), _su2(s)
        sd1, sd2 = _sd1(s), _sd2(s)
        s = [Q[r] + A[r] * s[r] + ec1[r] * ss1[r] + ec1d[r] * sd1[r]
             + ec2[r] * ss2[r] + ec2d[r] * sd2[r] for r in range(8)]

    fs1, fs2 = _su1(s), _su2(s)
    contrib = 0.0
    for r in range(8):
        d1 = s[r] - fs1[r]
        d2 = s[r] - fs2[r]
        en1 = jnp.sum(d1 * d1, axis=1, keepdims=True)
        en2 = jnp.sum(d2 * d2, axis=1, keepdims=True)
        own = (gidx[r] >= blk * T) & (gidx[r] < blk * T + T)
        contrib += jnp.sum(jnp.where(own, w1e[r] * en1 + w2e[r] * en2, 0.0))
    contrib *= 0.5
    prev = jnp.where(blk == 0, 0.0, eng_ref[0, 0, 0])
    eng_ref[...] = jnp.full_like(eng_ref, prev + contrib)

    for r in range(8):
        res = s[r] + h[r]
        mean = jnp.mean(res, axis=1, keepdims=True)
        cen = res - mean
        var = jnp.mean(cen * cen, axis=1, keepdims=True)
        norm = cen * jax.lax.rsqrt(var + 1e-5) * gamma_ref[...] + beta_ref[...]
        if W == T:
            o_scr[slot, r] = norm
        else:
            for off in (0, HALO // 8, 2 * (HALO // 8)):
                @pl.when(loc0p == off)
                def _(off=off, r=r, norm=norm):
                    o_scr[slot, r] = norm[off:off + TP]

    for c in out_copies(g, slot):
        c.start()

    @pl.when(g == G - 1)
    def _():
        for c in out_copies(g, slot):
            c.wait()
    if True:
        @pl.when((g == G - 1) & (G >= 2))
        def _():
            for c in out_copies(g - 1, 1 - slot):
                c.wait()


def kernel(hidden, attention_mask, W1, b1, W2, b2, Wq, bq, gamma, beta):
    del attention_mask
    B, L, H = hidden.shape
    T = BLK_T if L % BLK_T == 0 and L > BLK_T else L
    W = T + 2 * HALO if L > T else T
    nblk = L // T
    inner = W1.shape[1]

    f32 = jnp.float32
    grid = (B, nblk)
    out_shape = (
        jax.ShapeDtypeStruct((B, L // 8, 8, H), f32),
        jax.ShapeDtypeStruct((B, 8, 128), f32),
    )
    kern = functools.partial(_friction_kernel, L=L, T=T, W=W, NBLK=nblk)
    out4, eng = pl.pallas_call(
        kern,
        grid=grid,
        in_specs=[
            pl.BlockSpec(memory_space=pl.ANY),
            pl.BlockSpec((2, inner), lambda b, i: (0, 0)),
            pl.BlockSpec((1, inner), lambda b, i: (0, 0)),
            pl.BlockSpec((1, inner), lambda b, i: (0, 0)),
            pl.BlockSpec((1, 1), lambda b, i: (0, 0)),
            pl.BlockSpec((H, H), lambda b, i: (0, 0)),
            pl.BlockSpec((1, H), lambda b, i: (0, 0)),
            pl.BlockSpec((1, H), lambda b, i: (0, 0)),
            pl.BlockSpec((1, H), lambda b, i: (0, 0)),
        ],
        out_specs=(
            pl.BlockSpec(memory_space=pl.ANY),
            pl.BlockSpec((1, 8, 128), lambda b, i: (b, 0, 0)),
        ),
        out_shape=out_shape,
        scratch_shapes=[
            pltpu.VMEM((2, 8, W // 8, H), f32),
            pltpu.VMEM((2, 8, T // 8, H), f32),
            pltpu.SemaphoreType.DMA((2,)),
            pltpu.SemaphoreType.DMA((2,)),
        ],
        compiler_params=pltpu.CompilerParams(
            dimension_semantics=("arbitrary", "arbitrary"),
        ),
    )(
        hidden.reshape(B, L // 8, 8, H).astype(f32),
        W1.astype(f32),
        b1.reshape(1, inner).astype(f32),
        W2.reshape(1, inner).astype(f32),
        b2.reshape(1, 1).astype(f32),
        Wq.astype(f32),
        bq.reshape(1, H).astype(f32),
        gamma.reshape(1, H).astype(f32),
        beta.reshape(1, H).astype(f32),
    )
    return out4.reshape(B, L, H), eng[:, 0, 0]
